# baseline (speedup 1.0000x reference)
import functools

import jax
import jax.numpy as jnp
from jax import lax
from jax.experimental import pallas as pl
from jax.experimental.pallas import tpu as pltpu
from jax.experimental.pallas import tpu_sc as plsc

_C, _T, _H, _W = 3, 64, 256, 256
_TS = _T // 4
_IDX = (0, 4, 8, 12, 16, 21, 25, 29, 33, 37, 42, 46, 50, 54, 58, 63)

_NSLOW = _C * _TS
_NC, _NS = 2, 16
_NW = _NC * _NS
_HC = 128
_CPR = _H // _HC
_NBUF = 3


def _gather_body(x, slow, bufs, sems_in, sems_out):
    cid = lax.axis_index("c")
    sid = lax.axis_index("s")
    w = sid * _NC + cid
    for j0 in range(_NW):
        if j0 >= _NSLOW:
            continue
        rows = [j0] + ([j0 + _NW] if j0 + _NW < _NSLOW else [])

        @pl.when(w == j0)
        def _(rows=rows):
            chunks = []
            for j in rows:
                c = j // _TS
                t_src = _IDX[j % _TS]
                t_dst = j % _TS
                for k in range(_CPR):
                    h0 = k * _HC
                    chunks.append((
                        x.at[c, pl.ds(t_src, 1), pl.ds(h0, _HC)],
                        slow.at[c, pl.ds(t_dst, 1), pl.ds(h0, _HC)],
                    ))
            n = len(chunks)
            ins = []
            outs = []
            for k, (src, dst) in enumerate(chunks):
                b = k % _NBUF
                ins.append(pltpu.make_async_copy(src, bufs[b], sems_in[b]))
                outs.append(pltpu.make_async_copy(bufs[b], dst, sems_out[b]))
            ins[0].start()
            if n > 1:
                ins[1].start()
            for k in range(n):
                ins[k].wait()
                outs[k].start()
                if k + 2 < n:
                    if k + 2 >= _NBUF:
                        outs[k + 2 - _NBUF].wait()
                    ins[k + 2].start()
            for k in range(max(0, n - _NBUF), n):
                outs[k].wait()


@functools.partial(
    pl.kernel,
    out_type=jax.ShapeDtypeStruct((_C, _TS, _H, _W), jnp.float32),
    mesh=plsc.VectorSubcoreMesh(core_axis_name="c", subcore_axis_name="s"),
    scratch_types=(
        [pltpu.VMEM((1, _HC, _W), jnp.float32)] * _NBUF
        + [pltpu.SemaphoreType.DMA] * (2 * _NBUF)
    ),
)
def _gather_sc(x, slow, *scratch):
    bufs = scratch[:_NBUF]
    sems_in = scratch[_NBUF:2 * _NBUF]
    sems_out = scratch[2 * _NBUF:]
    _gather_body(x, slow, bufs, sems_in, sems_out)


def kernel(frames):
    slow = _gather_sc(frames)
    return (slow, frames)


# speedup vs baseline: 1.1523x; 1.1523x over previous
import functools

import jax
import jax.numpy as jnp
from jax import lax
from jax.experimental import pallas as pl
from jax.experimental.pallas import tpu as pltpu
from jax.experimental.pallas import tpu_sc as plsc

_C, _T, _H, _W = 3, 64, 256, 256
_TS = _T // 4
_IDX = (0, 4, 8, 12, 16, 21, 25, 29, 33, 37, 42, 46, 50, 54, 58, 63)

_NSLOW = _C * _TS
_NC, _NS = 2, 16
_NW = _NC * _NS
_HC = 128
_CPR = _H // _HC
_NBUF = 3


def _gather_body(x, slow, bufs, sems_in, sems_out):
    cid = lax.axis_index("c")
    sid = lax.axis_index("s")
    w = sid * _NC + cid
    for j0 in range(_NW):
        if j0 >= _NSLOW:
            continue
        rows = [j0] + ([j0 + _NW] if j0 + _NW < _NSLOW else [])

        @pl.when(w == j0)
        def _(rows=rows):
            chunks = []
            for j in rows:
                c = j // _TS
                t_src = _IDX[j % _TS]
                t_dst = j % _TS
                for k in range(_CPR):
                    h0 = k * _HC
                    chunks.append((
                        x.at[c, pl.ds(t_src, 1), pl.ds(h0, _HC)],
                        slow.at[c, pl.ds(t_dst, 1), pl.ds(h0, _HC)],
                    ))
            n = len(chunks)
            ins = []
            outs = []
            for k, (src, dst) in enumerate(chunks):
                b = k % _NBUF
                ins.append(pltpu.make_async_copy(src, bufs[b], sems_in[b]))
                outs.append(pltpu.make_async_copy(bufs[b], dst, sems_out[b]))
            ins[0].start()
            if n > 1:
                ins[1].start()
            for k in range(n):
                ins[k].wait()
                outs[k].start()
                if k + 2 < n:
                    if k + 2 >= _NBUF:
                        outs[k + 2 - _NBUF].wait()
                    ins[k + 2].start()
            for k in range(max(0, n - _NBUF), n):
                outs[k].wait()


@functools.partial(
    pl.kernel,
    out_type=jax.ShapeDtypeStruct((_C, _TS, _H, _W), jnp.float32),
    mesh=plsc.VectorSubcoreMesh(core_axis_name="c", subcore_axis_name="s"),
    scratch_types=(
        [pltpu.VMEM((1, _HC, _W), jnp.float32)] * _NBUF
        + [pltpu.SemaphoreType.DMA] * (2 * _NBUF)
    ),
)
def _gather_sc(x, slow, *scratch):
    bufs = scratch[:_NBUF]
    sems_in = scratch[_NBUF:2 * _NBUF]
    sems_out = scratch[2 * _NBUF:]
    _gather_body(x, slow, bufs, sems_in, sems_out)


_TCH = 32
_TNBUF = 4
_TRA = 2
_TCHUNKS = _C * (_T // _TCH)


def _copy_tc_body(x, out, *scratch):
    bufs = scratch[:_TNBUF]
    sems_in = scratch[_TNBUF:2 * _TNBUF]
    sems_out = scratch[2 * _TNBUF:]
    ins = []
    outs = []
    for k in range(_TCHUNKS):
        c = k // (_T // _TCH)
        t0 = (k % (_T // _TCH)) * _TCH
        b = k % _TNBUF
        ins.append(pltpu.make_async_copy(
            x.at[c, pl.ds(t0, _TCH)], bufs[b], sems_in[b]))
        outs.append(pltpu.make_async_copy(
            bufs[b], out.at[c, pl.ds(t0, _TCH)], sems_out[b]))
    for k in range(_TRA):
        ins[k].start()
    for k in range(_TCHUNKS):
        ins[k].wait()
        outs[k].start()
        nxt = k + _TRA
        if nxt < _TCHUNKS:
            if nxt - _TNBUF >= 0:
                outs[nxt - _TNBUF].wait()
            ins[nxt].start()
    for j in range(_TCHUNKS - _TNBUF, _TCHUNKS):
        outs[j].wait()


def _copy_tc(x):
    return pl.pallas_call(
        _copy_tc_body,
        in_specs=[pl.BlockSpec(memory_space=pl.ANY)],
        out_specs=pl.BlockSpec(memory_space=pl.ANY),
        out_shape=jax.ShapeDtypeStruct((_C, _T, _H, _W), jnp.float32),
        scratch_shapes=(
            [pltpu.VMEM((_TCH, _H, _W), jnp.float32)] * _TNBUF
            + [pltpu.SemaphoreType.DMA] * (2 * _TNBUF)
        ),
    )(x)


def kernel(frames):
    slow = _gather_sc(frames)
    fast = _copy_tc(frames)
    return (slow, fast)
